# initial kernel scaffold (unmeasured)
import jax
import jax.numpy as jnp
from jax import lax
from jax.experimental import pallas as pl
from jax.experimental.pallas import tpu as pltpu

N_DEV = 4
EPS = 1e-5


def kernel(x, gamma, beta):
    m, n_shard = x.shape
    n_global = n_shard * N_DEV

    def body(x_ref, g_ref, b_ref, o_ref, stats_ref, send_sems, recv_sems):
        my = lax.axis_index("i")

        barrier = pltpu.get_barrier_semaphore()
        for rel in (1, 2, 3):
            peer = lax.rem(my + rel, N_DEV)
            pl.semaphore_signal(
                barrier, inc=1,
                device_id=(peer,), device_id_type=pl.DeviceIdType.MESH,
            )
        pl.semaphore_wait(barrier, 3)

        xv = x_ref[:, :]
        stats_ref[0, 0, :] = jnp.sum(xv, axis=1)
        stats_ref[0, 1, :] = jnp.sum(xv * xv, axis=1)

        sends = []
        for rel in (1, 2, 3):
            peer = lax.rem(my + rel, N_DEV)
            rdma = pltpu.make_async_remote_copy(
                src_ref=stats_ref.at[0],
                dst_ref=stats_ref.at[N_DEV - rel],
                send_sem=send_sems.at[rel - 1],
                recv_sem=recv_sems.at[N_DEV - rel],
                device_id=(peer,),
                device_id_type=pl.DeviceIdType.MESH,
            )
            rdma.start()
            sends.append(rdma)

        for slot in (1, 2, 3):
            recv = pltpu.make_async_remote_copy(
                src_ref=stats_ref.at[0],
                dst_ref=stats_ref.at[slot],
                send_sem=send_sems.at[0],
                recv_sem=recv_sems.at[slot],
                device_id=(my,),
                device_id_type=pl.DeviceIdType.MESH,
            )
            recv.wait_recv()
        for rdma in sends:
            rdma.wait_send()

        tot = (
            stats_ref[0, :, :] + stats_ref[1, :, :]
            + stats_ref[2, :, :] + stats_ref[3, :, :]
        )
        tot_c = jnp.transpose(tot)
        mean = tot_c[:, 0:1] / n_global
        var = tot_c[:, 1:2] / n_global - mean * mean
        rstd = lax.rsqrt(var + EPS)

        o_ref[:, :] = (xv - mean) * rstd * g_ref[:] + b_ref[:]

    return pl.pallas_call(
        body,
        out_shape=jax.ShapeDtypeStruct((m, n_shard), jnp.float32),
        in_specs=[
            pl.BlockSpec(memory_space=pltpu.VMEM),
            pl.BlockSpec(memory_space=pltpu.VMEM),
            pl.BlockSpec(memory_space=pltpu.VMEM),
        ],
        out_specs=pl.BlockSpec(memory_space=pltpu.VMEM),
        scratch_shapes=[
            pltpu.VMEM((N_DEV, 2, m), jnp.float32),
            pltpu.SemaphoreType.DMA((3,)),
            pltpu.SemaphoreType.DMA((N_DEV,)),
        ],
        compiler_params=pltpu.CompilerParams(collective_id=0),
    )(x, gamma, beta)


# baseline (device time: 39512 ns/iter reference)
import jax
import jax.numpy as jnp
from jax import lax
from jax.experimental import pallas as pl
from jax.experimental.pallas import tpu as pltpu

N_DEV = 4
EPS = 1e-5


def kernel(x, gamma, beta):
    m, n_shard = x.shape
    n_global = n_shard * N_DEV

    def body(x_ref, g_ref, b_ref, o_ref, stats_ref, send_sems, recv_sems):
        my = lax.axis_index("i")

        barrier = pltpu.get_barrier_semaphore()
        for rel in (1, 2, 3):
            peer = lax.rem(my + rel, N_DEV)
            pl.semaphore_signal(
                barrier, inc=1,
                device_id=(peer,), device_id_type=pl.DeviceIdType.MESH,
            )
        pl.semaphore_wait(barrier, 3)

        xv = x_ref[:, :]
        stats_ref[0, 0, :] = jnp.sum(xv, axis=1)
        stats_ref[0, 1, :] = jnp.sum(xv * xv, axis=1)

        sends = []
        for rel in (1, 2, 3):
            peer = lax.rem(my + rel, N_DEV)
            rdma = pltpu.make_async_remote_copy(
                src_ref=stats_ref.at[0],
                dst_ref=stats_ref.at[N_DEV - rel],
                send_sem=send_sems.at[rel - 1],
                recv_sem=recv_sems.at[N_DEV - rel],
                device_id=(peer,),
                device_id_type=pl.DeviceIdType.MESH,
            )
            rdma.start()
            sends.append(rdma)

        for slot in (1, 2, 3):
            recv = pltpu.make_async_remote_copy(
                src_ref=stats_ref.at[0],
                dst_ref=stats_ref.at[slot],
                send_sem=send_sems.at[0],
                recv_sem=recv_sems.at[slot],
                device_id=(my,),
                device_id_type=pl.DeviceIdType.MESH,
            )
            recv.wait_recv()
        for rdma in sends:
            rdma.wait_send()

        tot = (
            stats_ref[0, :, :] + stats_ref[1, :, :]
            + stats_ref[2, :, :] + stats_ref[3, :, :]
        )
        tot_c = jnp.transpose(tot)
        mean = tot_c[:, 0:1] / n_global
        var = tot_c[:, 1:2] / n_global - mean * mean
        rstd = lax.rsqrt(var + EPS)

        o_ref[:, :] = (xv - mean) * rstd * g_ref[:] + b_ref[:]

    return pl.pallas_call(
        body,
        out_shape=jax.ShapeDtypeStruct((m, n_shard), jnp.float32),
        in_specs=[
            pl.BlockSpec(memory_space=pltpu.VMEM),
            pl.BlockSpec(memory_space=pltpu.VMEM),
            pl.BlockSpec(memory_space=pltpu.VMEM),
        ],
        out_specs=pl.BlockSpec(memory_space=pltpu.VMEM),
        scratch_shapes=[
            pltpu.VMEM((N_DEV, 2, m), jnp.float32),
            pltpu.SemaphoreType.DMA((3,)),
            pltpu.SemaphoreType.DMA((N_DEV,)),
        ],
        compiler_params=pltpu.CompilerParams(
            collective_id=0, vmem_limit_bytes=100 * 1024 * 1024
        ),
    )(x, gamma, beta)


# device time: 34253 ns/iter; 1.1535x vs baseline; 1.1535x over previous
import jax
import jax.numpy as jnp
from jax import lax
from jax.experimental import pallas as pl
from jax.experimental.pallas import tpu as pltpu

N_DEV = 4
EPS = 1e-5
N_CHUNKS = 8


def kernel(x, gamma, beta):
    m, n_shard = x.shape
    n_global = n_shard * N_DEV
    rows = m // N_CHUNKS

    def body(
        x_hbm, g_ref, b_ref, o_hbm,
        xbuf, stats_ref, in_sems, out_sems, send_sems, recv_sems,
    ):
        my = lax.axis_index("i")

        barrier = pltpu.get_barrier_semaphore()
        for rel in (1, 2, 3):
            peer = lax.rem(my + rel, N_DEV)
            pl.semaphore_signal(
                barrier, inc=1,
                device_id=(peer,), device_id_type=pl.DeviceIdType.MESH,
            )

        loads = []
        for b in range(N_CHUNKS):
            cp = pltpu.make_async_copy(
                x_hbm.at[pl.ds(b * rows, rows), :],
                xbuf.at[pl.ds(b * rows, rows), :],
                in_sems.at[b],
            )
            cp.start()
            loads.append(cp)

        sum_chunks, sq_chunks = [], []
        for b in range(N_CHUNKS):
            loads[b].wait()
            xb = xbuf[pl.ds(b * rows, rows), :]
            sum_chunks.append(jnp.sum(xb, axis=1, keepdims=True))
            sq_chunks.append(jnp.sum(xb * xb, axis=1, keepdims=True))
        pair = jnp.concatenate(
            [jnp.concatenate(sum_chunks, axis=0),
             jnp.concatenate(sq_chunks, axis=0)],
            axis=1,
        )
        stats_ref[0] = jnp.transpose(pair)

        pl.semaphore_wait(barrier, 3)
        sends = []
        for rel in (1, 2, 3):
            peer = lax.rem(my + rel, N_DEV)
            rdma = pltpu.make_async_remote_copy(
                src_ref=stats_ref.at[0],
                dst_ref=stats_ref.at[N_DEV - rel],
                send_sem=send_sems.at[rel - 1],
                recv_sem=recv_sems.at[N_DEV - rel],
                device_id=(peer,),
                device_id_type=pl.DeviceIdType.MESH,
            )
            rdma.start()
            sends.append(rdma)
        for slot in (1, 2, 3):
            recv = pltpu.make_async_remote_copy(
                src_ref=stats_ref.at[0],
                dst_ref=stats_ref.at[slot],
                send_sem=send_sems.at[0],
                recv_sem=recv_sems.at[slot],
                device_id=(my,),
                device_id_type=pl.DeviceIdType.MESH,
            )
            recv.wait_recv()

        tot = (
            stats_ref[0, :, :] + stats_ref[1, :, :]
            + stats_ref[2, :, :] + stats_ref[3, :, :]
        )
        tot_c = jnp.transpose(tot)
        mean = tot_c[:, 0:1] / n_global
        var = tot_c[:, 1:2] / n_global - mean * mean
        rstd = lax.rsqrt(var + EPS)

        gv = g_ref[:]
        bv = b_ref[:]
        stores = []
        for b in range(N_CHUNKS):
            sl = pl.ds(b * rows, rows)
            xb = xbuf[sl, :]
            mb = mean[b * rows:(b + 1) * rows, :]
            rb = rstd[b * rows:(b + 1) * rows, :]
            xbuf[sl, :] = (xb - mb) * rb * gv + bv
            cp = pltpu.make_async_copy(
                xbuf.at[sl, :], o_hbm.at[sl, :], out_sems.at[b]
            )
            cp.start()
            stores.append(cp)

        for cp in stores:
            cp.wait()
        for rdma in sends:
            rdma.wait_send()

    return pl.pallas_call(
        body,
        out_shape=jax.ShapeDtypeStruct((m, n_shard), jnp.float32),
        in_specs=[
            pl.BlockSpec(memory_space=pl.ANY),
            pl.BlockSpec(memory_space=pltpu.VMEM),
            pl.BlockSpec(memory_space=pltpu.VMEM),
        ],
        out_specs=pl.BlockSpec(memory_space=pl.ANY),
        scratch_shapes=[
            pltpu.VMEM((m, n_shard), jnp.float32),
            pltpu.VMEM((N_DEV, 2, m), jnp.float32),
            pltpu.SemaphoreType.DMA((N_CHUNKS,)),
            pltpu.SemaphoreType.DMA((N_CHUNKS,)),
            pltpu.SemaphoreType.DMA((3,)),
            pltpu.SemaphoreType.DMA((N_DEV,)),
        ],
        compiler_params=pltpu.CompilerParams(
            collective_id=0, vmem_limit_bytes=100 * 1024 * 1024
        ),
    )(x, gamma, beta)
